# baseline (speedup 1.0000x reference)
import functools

import jax
import jax.numpy as jnp
from jax import lax
from jax.experimental import pallas as pl
from jax.experimental.pallas import tpu as pltpu
from jax.experimental.pallas import tpu_sc as plsc


def _softargmax_one(sref, D):
    blk = sref[0]
    _, hblk, W = blk.shape
    x = blk.reshape(D, hblk * W)
    m = jnp.max(x, axis=0, keepdims=True)
    log2e = jnp.float32(1.4426950408889634)
    e = jnp.exp2(x * log2e - m * log2e)
    c = (D - 1) * 0.5
    i = jax.lax.broadcasted_iota(jnp.int32, (2, D), 0)
    j = jax.lax.broadcasted_iota(jnp.int32, (2, D), 1)
    w = jnp.where(i == 0, j.astype(jnp.float32) - c, 1.0)
    r = jax.lax.dot_general(
        w, e, (((1,), (0,)), ((), ())), preferred_element_type=jnp.float32
    )
    return (r[0] / r[1] + c).reshape(hblk, W)


def _tc_block(*refs, D, hblk):
    out_ref = refs[-1]
    for k, sref in enumerate(refs[:-1]):
        out_ref[0, 0, k * hblk : (k + 1) * hblk] = _softargmax_one(sref, D)


def _tc_softargmax(scores, h_tc):
    B, D, H, W = scores.shape
    hblk = 16
    S = h_tc // hblk
    grid = (B,)
    in_specs = [
        pl.BlockSpec((1, D, hblk, W), functools.partial(lambda k, b: (b, 0, k, 0), k))
        for k in range(S)
    ]
    return pl.pallas_call(
        functools.partial(_tc_block, D=D, hblk=hblk),
        grid=grid,
        in_specs=in_specs,
        out_specs=pl.BlockSpec((1, 1, h_tc, W), lambda b: (b, 0, 0, 0)),
        out_shape=jax.ShapeDtypeStruct((B, 1, h_tc, W), scores.dtype),
    )(*([scores] * S))


def _sc_softargmax(scores, h_tc, h_sc):
    B, D, H, W = scores.shape
    info = plsc.get_sparse_core_info()
    NC, NS, L = info.num_cores, info.num_subcores, info.num_lanes
    rows = B * h_sc
    rpw = rows // (NC * NS)
    U = 16

    @functools.partial(
        pl.kernel,
        out_type=jax.ShapeDtypeStruct((B, 1, h_sc, W), jnp.float32),
        mesh=plsc.VectorSubcoreMesh(core_axis_name="c", subcore_axis_name="s"),
        scratch_types=[
            pltpu.VMEM((D, W), jnp.float32),
            pltpu.VMEM((W,), jnp.float32),
        ],
    )
    def sc_kernel(scores_hbm, out_hbm, x_v, o_v):
        wid = lax.axis_index("s") * NC + lax.axis_index("c")
        for r in range(rpw):
            row = wid * rpw + r
            b = row // h_sc
            h = row % h_sc
            pltpu.sync_copy(scores_hbm.at[b, :, h_tc + h, :], x_v)
            for j in range(W // L):
                sl = pl.ds(j * L, L)

                def max_body(i, m):
                    for k in range(U):
                        m = jnp.maximum(m, x_v[i * U + k, sl])
                    return m

                m = lax.fori_loop(0, D // U, max_body, x_v[0, sl])

                def exp_body(i, carry):
                    num, den = carry
                    for k in range(U):
                        d_idx = i * U + k
                        e = jnp.exp(x_v[d_idx, sl] - m)
                        num = num + e * lax.convert_element_type(d_idx, jnp.float32)
                        den = den + e
                    return (num, den)

                z = jnp.zeros((L,), jnp.float32)
                num, den = lax.fori_loop(0, D // U, exp_body, (z, z))
                o_v[sl] = num / den
            pltpu.sync_copy(o_v, out_hbm.at[b, 0, h, :])

    return sc_kernel(scores)


def kernel(scores):
    B, D, H, W = scores.shape
    h_sc = 32
    h_tc = H - h_sc
    tc_out = _tc_softargmax(scores, h_tc)
    sc_out = _sc_softargmax(scores, h_tc, h_sc)
    return jnp.concatenate([tc_out, sc_out], axis=2)


# speedup vs baseline: 1.9047x; 1.9047x over previous
import functools

import jax
import jax.numpy as jnp
from jax.experimental import pallas as pl


def _softargmax_slab(parts, D, SD, out_ref, lo, hi):
    Dq = D // SD
    xs = []
    for p in parts:
        blk = p[0]
        _, hblk, W = blk.shape
        xs.append(blk.reshape(Dq, hblk * W))
    m = xs[0]
    for x in xs[1:]:
        m = jnp.maximum(m, x)
    m = jnp.max(m, axis=0, keepdims=True)
    log2e = jnp.float32(1.4426950408889634)
    mscaled = m * log2e
    c = (D - 1) * 0.5
    i = jax.lax.broadcasted_iota(jnp.int32, (2, Dq), 0)
    j = jax.lax.broadcasted_iota(jnp.int32, (2, Dq), 1)
    num = None
    den = None
    for q, x in enumerate(xs):
        e = jnp.exp2(x * log2e - mscaled)
        w = jnp.where(i == 0, j.astype(jnp.float32) + (q * Dq - c), 1.0)
        r = jax.lax.dot_general(
            w, e, (((1,), (0,)), ((), ())), preferred_element_type=jnp.float32
        )
        num = r[0] if num is None else num + r[0]
        den = r[1] if den is None else den + r[1]
    out_ref[0, 0, lo:hi] = (num / den + c).reshape(hblk, W)


def _tc_block(*refs, D, hblk, SD):
    out_ref = refs[-1]
    ins = refs[:-1]
    SH = len(ins) // SD
    for k in range(SH):
        parts = ins[k * SD : (k + 1) * SD]
        _softargmax_slab(parts, D, SD, out_ref, k * hblk, (k + 1) * hblk)


def kernel(scores):
    B, D, H, W = scores.shape
    SH = 4
    SD = 4
    hblk = 8
    grid = (B, H // (SH * hblk))
    in_specs = [
        pl.BlockSpec(
            (1, D // SD, hblk, W),
            functools.partial(lambda k, q, b, h: (b, q, SH * h + k, 0), k, q),
        )
        for k in range(SH)
        for q in range(SD)
    ]
    out = pl.pallas_call(
        functools.partial(_tc_block, D=D, hblk=hblk, SD=SD),
        grid=grid,
        in_specs=in_specs,
        out_specs=pl.BlockSpec((1, 1, SH * hblk, W), lambda b, h: (b, 0, h, 0)),
        out_shape=jax.ShapeDtypeStruct((B, 1, H, W), scores.dtype),
    )(*([scores] * (SH * SD)))
    return out


# speedup vs baseline: 2.2488x; 1.1807x over previous
import functools

import jax
import jax.numpy as jnp
from jax.experimental import pallas as pl


def _softargmax_slab(parts, D, SD, out_ref, lo, hi):
    Dq = D // SD
    xs = []
    for p in parts:
        blk = p[0]
        _, hblk, W = blk.shape
        xs.append(blk.reshape(Dq, hblk * W))
    m = xs[0]
    for x in xs[1:]:
        m = jnp.maximum(m, x)
    m = jnp.max(m, axis=0, keepdims=True)
    log2e = jnp.float32(1.4426950408889634)
    mscaled = m * log2e
    c = (D - 1) * 0.5
    i = jax.lax.broadcasted_iota(jnp.int32, (2, Dq), 0)
    j = jax.lax.broadcasted_iota(jnp.int32, (2, Dq), 1)
    num = None
    den = None
    for q, x in enumerate(xs):
        e = jnp.exp2(x * log2e - mscaled)
        w = jnp.where(i == 0, j.astype(jnp.float32) + (q * Dq - c), 1.0)
        r = jax.lax.dot_general(
            w, e, (((1,), (0,)), ((), ())), preferred_element_type=jnp.float32
        )
        num = r[0] if num is None else num + r[0]
        den = r[1] if den is None else den + r[1]
    out_ref[0, 0, lo:hi] = (num / den + c).reshape(hblk, W)


def _tc_block(*refs, D, hblk, SD):
    out_ref = refs[-1]
    ins = refs[:-1]
    SH = len(ins) // SD
    for k in range(SH):
        parts = ins[k * SD : (k + 1) * SD]
        _softargmax_slab(parts, D, SD, out_ref, k * hblk, (k + 1) * hblk)


def kernel(scores):
    B, D, H, W = scores.shape
    SH = 8
    SD = 2
    hblk = 8
    grid = (B, H // (SH * hblk))
    in_specs = [
        pl.BlockSpec(
            (1, D // SD, hblk, W),
            functools.partial(lambda k, q, b, h: (b, q, SH * h + k, 0), k, q),
        )
        for k in range(SH)
        for q in range(SD)
    ]
    out = pl.pallas_call(
        functools.partial(_tc_block, D=D, hblk=hblk, SD=SD),
        grid=grid,
        in_specs=in_specs,
        out_specs=pl.BlockSpec((1, 1, SH * hblk, W), lambda b, h: (b, 0, h, 0)),
        out_shape=jax.ShapeDtypeStruct((B, 1, H, W), scores.dtype),
    )(*([scores] * (SH * SD)))
    return out


# speedup vs baseline: 2.4896x; 1.1071x over previous
import functools

import jax
import jax.numpy as jnp
from jax.experimental import pallas as pl


def _softargmax_one(sref, D):
    blk = sref[0]
    _, hblk, W = blk.shape
    x = blk.reshape(D, hblk * W)
    m = jnp.max(x, axis=0, keepdims=True)
    log2e = jnp.float32(1.4426950408889634)
    e = jnp.exp2(x * log2e - m * log2e)
    c = (D - 1) * 0.5
    i = jax.lax.broadcasted_iota(jnp.int32, (2, D), 0)
    j = jax.lax.broadcasted_iota(jnp.int32, (2, D), 1)
    w = jnp.where(i == 0, j.astype(jnp.float32) - c, 1.0)
    r = jax.lax.dot_general(
        w, e, (((1,), (0,)), ((), ())), preferred_element_type=jnp.float32
    )
    return (r[0] / r[1] + c).reshape(hblk, W)


def _softargmax_block(*refs, D, hblk):
    out_ref = refs[-1]
    for k, sref in enumerate(refs[:-1]):
        out_ref[0, 0, k * hblk : (k + 1) * hblk] = _softargmax_one(sref, D)


def kernel(scores):
    B, D, H, W = scores.shape
    S = 16
    hblk = 8
    grid = (B, H // (S * hblk))
    in_specs = [
        pl.BlockSpec((1, D, hblk, W), functools.partial(lambda k, b, h: (b, 0, S * h + k, 0), k))
        for k in range(S)
    ]
    out = pl.pallas_call(
        functools.partial(_softargmax_block, D=D, hblk=hblk),
        grid=grid,
        in_specs=in_specs,
        out_specs=pl.BlockSpec((1, 1, S * hblk, W), lambda b, h: (b, 0, h, 0)),
        out_shape=jax.ShapeDtypeStruct((B, 1, H, W), scores.dtype),
    )(*([scores] * S))
    return out


# speedup vs baseline: 2.5034x; 1.0056x over previous
import functools

import jax
import jax.numpy as jnp
from jax.experimental import pallas as pl
from jax.experimental.pallas import tpu as pltpu


def _softargmax_one(sref, D):
    blk = sref[0]
    _, hblk, W = blk.shape
    x = blk.reshape(D, hblk * W)
    m = jnp.max(x, axis=0, keepdims=True)
    log2e = jnp.float32(1.4426950408889634)
    e = jnp.exp2(x * log2e - m * log2e)
    c = (D - 1) * 0.5
    i = jax.lax.broadcasted_iota(jnp.int32, (2, D), 0)
    j = jax.lax.broadcasted_iota(jnp.int32, (2, D), 1)
    w = jnp.where(i == 0, j.astype(jnp.float32) - c, 1.0)
    r = jax.lax.dot_general(
        w, e, (((1,), (0,)), ((), ())), preferred_element_type=jnp.float32
    )
    return (r[0] / r[1] + c).reshape(hblk, W)


def _softargmax_block(*refs, D, hblk):
    out_ref = refs[-1]
    for k, sref in enumerate(refs[:-1]):
        out_ref[0, 0, k * hblk : (k + 1) * hblk] = _softargmax_one(sref, D)


def kernel(scores):
    B, D, H, W = scores.shape
    S = 16
    hblk = 8
    grid = (B, H // (S * hblk))
    in_specs = [
        pl.BlockSpec((1, D, hblk, W), functools.partial(lambda k, b, h: (b, 0, S * h + k, 0), k))
        for k in range(S)
    ]
    out = pl.pallas_call(
        functools.partial(_softargmax_block, D=D, hblk=hblk),
        grid=grid,
        in_specs=in_specs,
        out_specs=pl.BlockSpec((1, 1, S * hblk, W), lambda b, h: (b, 0, h, 0)),
        out_shape=jax.ShapeDtypeStruct((B, 1, H, W), scores.dtype),
        compiler_params=pltpu.CompilerParams(dimension_semantics=("parallel", "arbitrary")),
    )(*([scores] * S))
    return out
